# baseline (speedup 1.0000x reference)
import jax
import jax.numpy as jnp
import numpy as np
from jax.experimental import pallas as pl
from jax.experimental.pallas import tpu as pltpu

_STEPS = (1, 2, 3, 4, 5, 6, 8, 10, 13, 16)
_L = 32
_K = 4096
_B = 8
_M = 256


def _round8(n):
    return ((n + 7) // 8) * 8


_NPAD = tuple(_round8(s * s) for s in _STEPS)


def _conv2d(x, w, b, stride=1, padding=0):
    out = jax.lax.conv_general_dilated(
        x, w, (stride, stride), [(padding, padding), (padding, padding)],
        dimension_numbers=('NCHW', 'OIHW', 'NCHW'))
    return out + b[None, :, None, None]


def _resize_consts():
    eye16 = jnp.eye(16, dtype=jnp.float32)
    ds, u9s = [], []
    for s, npad in zip(_STEPS, _NPAD):
        n = s * s
        r = jax.image.resize(eye16, (s, 16), 'bilinear')
        d = r[:, None, :, None] * r[None, :, None, :]
        d = d.reshape(n, 256)
        d = jnp.zeros((npad, 256), jnp.float32).at[:n].set(d)
        u = jax.image.resize(jnp.eye(s, dtype=jnp.float32), (16, s), 'bilinear')
        ufull = (u[:, None, :, None] * u[None, :, None, :])
        up = jnp.pad(ufull, ((1, 1), (1, 1), (0, 0), (0, 0)))
        u9 = jnp.stack([
            up[dy:dy + 16, dx:dx + 16].reshape(256, n)
            for dy in range(3) for dx in range(3)])
        u9 = jnp.zeros((9, 256, npad), jnp.float32).at[:, :, :n].set(u9)
        ds.append(d)
        u9s.append(u9)
    return ds, u9s


def _vq_body(xm_ref, ct_ref, w_ref, b_ref, *refs):
    nd = len(_STEPS)
    d_refs = refs[:nd]
    u9_refs = refs[nd:2 * nd]
    f_ref = refs[2 * nd]
    idx_refs = refs[2 * nd + 1:3 * nd + 1]
    loss_ref = refs[3 * nd + 1]

    bprog = pl.program_id(0)
    x_cur = xm_ref[0]
    ct = ct_ref[...]
    ct16 = ct.astype(jnp.bfloat16)
    c2 = jnp.sum(ct * ct, axis=0, keepdims=True)
    f_acc = jnp.zeros((_M, _L), jnp.float32)
    loss = jnp.zeros((1, 1), jnp.float32)

    for i, s in enumerate(_STEPS):
        n = s * s
        npad = _NPAD[i]
        dmat = d_refs[i][...]
        z = jnp.dot(dmat, x_cur, preferred_element_type=jnp.float32, precision=jax.lax.Precision.HIGHEST)
        g = jnp.dot(z.astype(jnp.bfloat16), ct16,
                    preferred_element_type=jnp.float32)
        z2 = jnp.sum(z * z, axis=1, keepdims=True)
        score = z2 - 2.0 * g + c2
        mn = jnp.min(score, axis=1, keepdims=True)
        kio = jax.lax.broadcasted_iota(jnp.int32, (npad, _K), 1)
        idx = jnp.min(jnp.where(score == mn, kio, _K), axis=1, keepdims=True)
        onehot = (kio == idx).astype(jnp.float32)
        q = jax.lax.dot_general(onehot, ct, (((1,), (1,)), ((), ())),
                                preferred_element_type=jnp.float32, precision=jax.lax.Precision.HIGHEST)
        err = q - z
        rmask = jax.lax.broadcasted_iota(jnp.int32, (npad, _L), 0) < n
        err = jnp.where(rmask, err, 0.0)
        sq = jnp.sum(jnp.sum(err * err, axis=1, keepdims=True),
                     axis=0, keepdims=True)
        loss = loss + sq * (1.25 / (_B * n * _L))
        q_st = z + (q - z)
        r = jnp.zeros((_M, _L), jnp.float32) + b_ref[i, :][None, :]
        for t in range(9):
            u9q = jnp.dot(u9_refs[i][t], q_st,
                          preferred_element_type=jnp.float32,
                          precision=jax.lax.Precision.HIGHEST)
            r = r + jnp.dot(u9q.astype(jnp.bfloat16), w_ref[i, t],
                            preferred_element_type=jnp.float32)
        x_cur = x_cur - r
        f_acc = f_acc + r
        idx_refs[i][...] = idx[None]

    f_ref[0] = f_acc

    @pl.when(bprog == 0)
    def _():
        loss_ref[...] = loss

    @pl.when(bprog != 0)
    def _():
        loss_ref[...] = loss_ref[...] + loss


def _vq_stage(xm, ct, wall, ball, dmats, u9mats):
    nd = len(_STEPS)
    whole = lambda a: pl.BlockSpec(a.shape, lambda b: (0,) * a.ndim)
    in_specs = [pl.BlockSpec((1, _M, _L), lambda b: (b, 0, 0)),
                whole(ct), whole(wall), whole(ball)]
    in_specs += [whole(d) for d in dmats]
    in_specs += [whole(u) for u in u9mats]
    out_shape = [jax.ShapeDtypeStruct((_B, _M, _L), jnp.float32)]
    out_shape += [jax.ShapeDtypeStruct((_B, npad, 1), jnp.int32) for npad in _NPAD]
    out_shape += [jax.ShapeDtypeStruct((1, 1), jnp.float32)]
    out_specs = [pl.BlockSpec((1, _M, _L), lambda b: (b, 0, 0))]
    out_specs += [pl.BlockSpec((1, npad, 1), lambda b: (b, 0, 0)) for npad in _NPAD]
    out_specs += [pl.BlockSpec((1, 1), lambda b: (0, 0))]
    return pl.pallas_call(
        _vq_body,
        grid=(_B,),
        in_specs=in_specs,
        out_specs=out_specs,
        out_shape=out_shape,
        compiler_params=pltpu.CompilerParams(
            dimension_semantics=('arbitrary',)),
    )(xm, ct, wall, ball, *dmats, *u9mats)


def kernel(x, params):
    p = params
    h = x
    for i in range(4):
        h = jax.nn.relu(_conv2d(h, p['enc_w%d' % i], p['enc_b%d' % i],
                                stride=2, padding=1))
    h = _conv2d(h, p['enc_wout'], p['enc_bout'])
    h = _conv2d(h, p['quant_w'], p['quant_b'])
    x_lat = h.astype(jnp.float32)

    xm = x_lat.transpose(0, 2, 3, 1).reshape(_B, _M, _L)
    ct = p['codebook'].T
    wall = jnp.stack([
        jnp.stack([p['res_w%d' % i][:, :, t // 3, t % 3].T for t in range(9)])
        for i in range(len(_STEPS))]).astype(jnp.bfloat16)
    ball = jnp.zeros((16, _L), jnp.float32).at[:len(_STEPS)].set(
        jnp.stack([p['res_b%d' % i] for i in range(len(_STEPS))]))
    dmats, u9mats = _resize_consts()

    outs = _vq_stage(xm, ct, wall, ball, dmats, u9mats)
    f_flat = outs[0]
    idx_pads = outs[1:1 + len(_STEPS)]
    vqloss = outs[-1][0, 0]

    token_maps = tuple(idx_pads[i][:, :s * s, 0]
                       for i, s in enumerate(_STEPS))
    f = f_flat.reshape(_B, 16, 16, _L).transpose(0, 3, 1, 2)

    d = jax.nn.relu(_conv2d(f, p['dec_win'], p['dec_bin']))
    for i in range(4):
        d = jax.image.resize(d, (d.shape[0], d.shape[1],
                                 d.shape[2] * 2, d.shape[3] * 2), 'bilinear')
        d = jax.nn.relu(_conv2d(d, p['dec_w%d' % i], p['dec_b%d' % i],
                                padding=1))
    d = _conv2d(d, p['dec_wout'], p['dec_bout'], padding=1)
    return d, token_maps, vqloss


# speedup vs baseline: 1.0060x; 1.0060x over previous
import jax
import jax.numpy as jnp
import numpy as np
from jax.experimental import pallas as pl
from jax.experimental.pallas import tpu as pltpu

_STEPS = (1, 2, 3, 4, 5, 6, 8, 10, 13, 16)
_L = 32
_K = 4096
_B = 8
_M = 256


def _round8(n):
    return ((n + 7) // 8) * 8


_NPAD = tuple(_round8(s * s) for s in _STEPS)


def _conv2d(x, w, b, stride=1, padding=0):
    out = jax.lax.conv_general_dilated(
        x, w, (stride, stride), [(padding, padding), (padding, padding)],
        dimension_numbers=('NCHW', 'OIHW', 'NCHW'))
    return out + b[None, :, None, None]


def _resize_consts():
    eye16 = jnp.eye(16, dtype=jnp.float32)
    ds, u9s = [], []
    for s, npad in zip(_STEPS, _NPAD):
        n = s * s
        r = jax.image.resize(eye16, (s, 16), 'bilinear')
        d = r[:, None, :, None] * r[None, :, None, :]
        d = d.reshape(n, 256)
        d = jnp.zeros((npad, 256), jnp.float32).at[:n].set(d)
        u = jax.image.resize(jnp.eye(s, dtype=jnp.float32), (16, s), 'bilinear')
        ufull = (u[:, None, :, None] * u[None, :, None, :])
        up = jnp.pad(ufull, ((1, 1), (1, 1), (0, 0), (0, 0)))
        u9 = jnp.stack([
            up[dy:dy + 16, dx:dx + 16].reshape(256, n)
            for dy in range(3) for dx in range(3)])
        u9 = jnp.zeros((9, 256, npad), jnp.float32).at[:, :, :n].set(u9)
        ds.append(d)
        u9s.append(u9)
    return ds, u9s


def _vq_body(xm_ref, ct_ref, w_ref, b_ref, *refs):
    nd = len(_STEPS)
    d_refs = refs[:nd]
    u9_refs = refs[nd:2 * nd]
    f_ref = refs[2 * nd]
    idx_refs = refs[2 * nd + 1:3 * nd + 1]
    loss_ref = refs[3 * nd + 1]

    bprog = pl.program_id(0)
    x_cur = xm_ref[0]
    ct = ct_ref[...]
    ct16 = ct.astype(jnp.bfloat16)
    c2 = jnp.sum(ct * ct, axis=0, keepdims=True)
    f_acc = jnp.zeros((_M, _L), jnp.float32)
    loss = jnp.zeros((1, 1), jnp.float32)

    for i, s in enumerate(_STEPS):
        n = s * s
        npad = _NPAD[i]
        dmat = d_refs[i][...]
        z = jnp.dot(dmat, x_cur, preferred_element_type=jnp.float32, precision=jax.lax.Precision.HIGHEST)
        g = jnp.dot(z.astype(jnp.bfloat16), ct16,
                    preferred_element_type=jnp.float32)
        z2 = jnp.sum(z * z, axis=1, keepdims=True)
        score = z2 - 2.0 * g + c2
        mn = jnp.min(score, axis=1, keepdims=True)
        kio = jax.lax.broadcasted_iota(jnp.int32, (npad, _K), 1)
        idx = jnp.min(jnp.where(score == mn, kio, _K), axis=1, keepdims=True)
        onehot = (kio == idx).astype(jnp.float32)
        q = jax.lax.dot_general(onehot, ct, (((1,), (1,)), ((), ())),
                                preferred_element_type=jnp.float32, precision=jax.lax.Precision.HIGHEST)
        err = q - z
        rmask = jax.lax.broadcasted_iota(jnp.int32, (npad, _L), 0) < n
        err = jnp.where(rmask, err, 0.0)
        sq = jnp.sum(jnp.sum(err * err, axis=1, keepdims=True),
                     axis=0, keepdims=True)
        loss = loss + sq * (1.25 / (_B * n * _L))
        q_st = z + (q - z)
        u9stack = u9_refs[i][...].reshape(9 * _M, npad)
        u9q = jnp.dot(u9stack, q_st, preferred_element_type=jnp.float32,
                      precision=jax.lax.Precision.HIGHEST)
        u9q16 = jnp.concatenate(
            [u9q[t * _M:(t + 1) * _M] for t in range(9)],
            axis=1).astype(jnp.bfloat16)
        r = jnp.dot(u9q16, w_ref[i].reshape(9 * _L, _L),
                    preferred_element_type=jnp.float32)
        r = r + b_ref[i, :][None, :]
        x_cur = x_cur - r
        f_acc = f_acc + r
        idx_refs[i][...] = idx[None]

    f_ref[0] = f_acc

    @pl.when(bprog == 0)
    def _():
        loss_ref[...] = loss

    @pl.when(bprog != 0)
    def _():
        loss_ref[...] = loss_ref[...] + loss


def _vq_stage(xm, ct, wall, ball, dmats, u9mats):
    nd = len(_STEPS)
    whole = lambda a: pl.BlockSpec(a.shape, lambda b: (0,) * a.ndim)
    in_specs = [pl.BlockSpec((1, _M, _L), lambda b: (b, 0, 0)),
                whole(ct), whole(wall), whole(ball)]
    in_specs += [whole(d) for d in dmats]
    in_specs += [whole(u) for u in u9mats]
    out_shape = [jax.ShapeDtypeStruct((_B, _M, _L), jnp.float32)]
    out_shape += [jax.ShapeDtypeStruct((_B, npad, 1), jnp.int32) for npad in _NPAD]
    out_shape += [jax.ShapeDtypeStruct((1, 1), jnp.float32)]
    out_specs = [pl.BlockSpec((1, _M, _L), lambda b: (b, 0, 0))]
    out_specs += [pl.BlockSpec((1, npad, 1), lambda b: (b, 0, 0)) for npad in _NPAD]
    out_specs += [pl.BlockSpec((1, 1), lambda b: (0, 0))]
    return pl.pallas_call(
        _vq_body,
        grid=(_B,),
        in_specs=in_specs,
        out_specs=out_specs,
        out_shape=out_shape,
        compiler_params=pltpu.CompilerParams(
            dimension_semantics=('arbitrary',)),
    )(xm, ct, wall, ball, *dmats, *u9mats)


def kernel(x, params):
    p = params
    h = x
    for i in range(4):
        h = jax.nn.relu(_conv2d(h, p['enc_w%d' % i], p['enc_b%d' % i],
                                stride=2, padding=1))
    h = _conv2d(h, p['enc_wout'], p['enc_bout'])
    h = _conv2d(h, p['quant_w'], p['quant_b'])
    x_lat = h.astype(jnp.float32)

    xm = x_lat.transpose(0, 2, 3, 1).reshape(_B, _M, _L)
    ct = p['codebook'].T
    wall = jnp.stack([
        jnp.stack([p['res_w%d' % i][:, :, t // 3, t % 3].T for t in range(9)])
        for i in range(len(_STEPS))]).astype(jnp.bfloat16)
    ball = jnp.zeros((16, _L), jnp.float32).at[:len(_STEPS)].set(
        jnp.stack([p['res_b%d' % i] for i in range(len(_STEPS))]))
    dmats, u9mats = _resize_consts()

    outs = _vq_stage(xm, ct, wall, ball, dmats, u9mats)
    f_flat = outs[0]
    idx_pads = outs[1:1 + len(_STEPS)]
    vqloss = outs[-1][0, 0]

    token_maps = tuple(idx_pads[i][:, :s * s, 0]
                       for i, s in enumerate(_STEPS))
    f = f_flat.reshape(_B, 16, 16, _L).transpose(0, 3, 1, 2)

    d = jax.nn.relu(_conv2d(f, p['dec_win'], p['dec_bin']))
    for i in range(4):
        d = jax.image.resize(d, (d.shape[0], d.shape[1],
                                 d.shape[2] * 2, d.shape[3] * 2), 'bilinear')
        d = jax.nn.relu(_conv2d(d, p['dec_w%d' % i], p['dec_b%d' % i],
                                padding=1))
    d = _conv2d(d, p['dec_wout'], p['dec_bout'], padding=1)
    return d, token_maps, vqloss


# speedup vs baseline: 1.0553x; 1.0490x over previous
import jax
import jax.numpy as jnp
from jax.experimental import pallas as pl
from jax.experimental.pallas import tpu as pltpu

_STEPS = (1, 2, 3, 4, 5, 6, 8, 10, 13, 16)
_L = 32
_K = 4096
_B = 8
_M = 256
_BC = _B * _L


def _round8(n):
    return ((n + 7) // 8) * 8


_NPAD = tuple(_round8(s * s) for s in _STEPS)


def _conv2d(x, w, b, stride=1, padding=0):
    out = jax.lax.conv_general_dilated(
        x, w, (stride, stride), [(padding, padding), (padding, padding)],
        dimension_numbers=('NCHW', 'OIHW', 'NCHW'))
    return out + b[None, :, None, None]


def _resize_consts():
    eye16 = jnp.eye(16, dtype=jnp.float32)
    ds, u9s = [], []
    for s, npad in zip(_STEPS, _NPAD):
        n = s * s
        r = jax.image.resize(eye16, (s, 16), 'bilinear')
        d = r[:, None, :, None] * r[None, :, None, :]
        d = d.reshape(n, 256)
        d = jnp.zeros((npad, 256), jnp.float32).at[:n].set(d)
        u = jax.image.resize(jnp.eye(s, dtype=jnp.float32), (16, s), 'bilinear')
        ufull = (u[:, None, :, None] * u[None, :, None, :])
        up = jnp.pad(ufull, ((1, 1), (1, 1), (0, 0), (0, 0)))
        u9 = jnp.stack([
            up[dy:dy + 16, dx:dx + 16].reshape(256, n)
            for dy in range(3) for dx in range(3)])
        u9 = jnp.zeros((9, 256, npad), jnp.float32).at[:, :, :n].set(u9)
        ds.append(d)
        u9s.append(u9.reshape(9 * 256, npad))
    return ds, u9s


def _vq_body(xm_ref, ct_ref, w_ref, b_ref, *refs):
    nd = len(_STEPS)
    d_refs = refs[:nd]
    u9_refs = refs[nd:2 * nd]
    f_ref = refs[2 * nd]
    idx_refs = refs[2 * nd + 1:3 * nd + 1]
    loss_ref = refs[3 * nd + 1]

    x_cur = xm_ref[...]
    ct = ct_ref[...]
    ct16 = ct.astype(jnp.bfloat16)
    c2 = jnp.sum(ct * ct, axis=0, keepdims=True)
    f_acc = jnp.zeros((_M, _BC), jnp.float32)
    loss = jnp.zeros((1, 1), jnp.float32)

    for i, s in enumerate(_STEPS):
        n = s * s
        npad = _NPAD[i]
        rows = _B * npad
        dmat = d_refs[i][...]
        zbc = jnp.dot(dmat, x_cur, preferred_element_type=jnp.float32,
                      precision=jax.lax.Precision.HIGHEST)
        zr = zbc.reshape(npad, _B, _L).transpose(1, 0, 2).reshape(rows, _L)
        n_chunks = -(-rows // 512)
        while rows % n_chunks or (rows // n_chunks) % 8:
            n_chunks += 1
        ch = rows // n_chunks
        idx_parts, q_parts = [], []
        for ci in range(n_chunks):
            zc = zr[ci * ch:(ci + 1) * ch]
            g = jnp.dot(zc.astype(jnp.bfloat16), ct16,
                        preferred_element_type=jnp.float32)
            z2 = jnp.sum(zc * zc, axis=1, keepdims=True)
            score = z2 - 2.0 * g + c2
            mn = jnp.min(score, axis=1, keepdims=True)
            kio = jax.lax.broadcasted_iota(jnp.int32, (ch, _K), 1)
            idx_parts.append(jnp.min(jnp.where(score == mn, kio, _K),
                                     axis=1, keepdims=True))
            onehot = (kio == idx_parts[-1]).astype(jnp.float32)
            q_parts.append(jax.lax.dot_general(
                onehot, ct, (((1,), (1,)), ((), ())),
                preferred_element_type=jnp.float32,
                precision=jax.lax.Precision.HIGHEST))
        idx = jnp.concatenate(idx_parts, axis=0) if n_chunks > 1 else idx_parts[0]
        q = jnp.concatenate(q_parts, axis=0) if n_chunks > 1 else q_parts[0]
        err = q - zr
        rmask = (jax.lax.broadcasted_iota(jnp.int32, (rows, _L), 0) % npad) < n
        err = jnp.where(rmask, err, 0.0)
        sq = jnp.sum(jnp.sum(err * err, axis=1, keepdims=True),
                     axis=0, keepdims=True)
        loss = loss + sq * (1.25 / (_B * n * _L))
        q_st = zr + (q - zr)
        qbc = q_st.reshape(_B, npad, _L).transpose(1, 0, 2).reshape(npad, _BC)
        u9q = jnp.dot(u9_refs[i][...], qbc,
                      preferred_element_type=jnp.float32,
                      precision=jax.lax.Precision.HIGHEST)
        u9q16 = jnp.concatenate(
            [u9q[t * _M:(t + 1) * _M] for t in range(9)],
            axis=1).astype(jnp.bfloat16)
        r = jnp.dot(u9q16, w_ref[i],
                    preferred_element_type=jnp.float32)
        r = r + b_ref[i, :][None, :]
        x_cur = x_cur - r
        f_acc = f_acc + r
        idx_refs[i][...] = idx

    f_ref[...] = f_acc
    loss_ref[...] = loss


def _vq_stage(xm, ct, wbd, ballbc, dmats, u9mats):
    whole = lambda a: pl.BlockSpec(a.shape, lambda: (0,) * a.ndim)
    in_specs = [whole(xm), whole(ct), whole(wbd), whole(ballbc)]
    in_specs += [whole(d) for d in dmats]
    in_specs += [whole(u) for u in u9mats]
    out_shape = [jax.ShapeDtypeStruct((_M, _BC), jnp.float32)]
    out_shape += [jax.ShapeDtypeStruct((_B * npad, 1), jnp.int32)
                  for npad in _NPAD]
    out_shape += [jax.ShapeDtypeStruct((1, 1), jnp.float32)]
    out_specs = [pl.BlockSpec((_M, _BC), lambda: (0, 0))]
    out_specs += [pl.BlockSpec((_B * npad, 1), lambda: (0, 0))
                  for npad in _NPAD]
    out_specs += [pl.BlockSpec((1, 1), lambda: (0, 0))]
    return pl.pallas_call(
        _vq_body,
        in_specs=in_specs,
        out_specs=out_specs,
        out_shape=out_shape,
    )(xm, ct, wbd, ballbc, *dmats, *u9mats)


def kernel(x, params):
    p = params
    h = x
    for i in range(4):
        h = jax.nn.relu(_conv2d(h, p['enc_w%d' % i], p['enc_b%d' % i],
                                stride=2, padding=1))
    h = _conv2d(h, p['enc_wout'], p['enc_bout'])
    h = _conv2d(h, p['quant_w'], p['quant_b'])
    x_lat = h.astype(jnp.float32)

    xm = x_lat.transpose(2, 3, 0, 1).reshape(_M, _BC)
    ct = p['codebook'].T
    wtap = jnp.stack([
        jnp.stack([p['res_w%d' % i][:, :, t // 3, t % 3].T for t in range(9)])
        for i in range(len(_STEPS))])
    eyeb = jnp.eye(_B, dtype=jnp.float32)
    wbd = (wtap[:, :, None, :, None, :] * eyeb[None, None, :, None, :, None])
    wbd = wbd.reshape(len(_STEPS), 9 * _BC, _BC).astype(jnp.bfloat16)
    ballbc = jnp.zeros((16, _BC), jnp.float32).at[:len(_STEPS)].set(
        jnp.stack([jnp.tile(p['res_b%d' % i], _B)
                   for i in range(len(_STEPS))]))
    dmats, u9mats = _resize_consts()

    outs = _vq_stage(xm, ct, wbd, ballbc, dmats, u9mats)
    f_flat = outs[0]
    idx_pads = outs[1:1 + len(_STEPS)]
    vqloss = outs[-1][0, 0]

    token_maps = tuple(
        idx_pads[i].reshape(_B, _NPAD[i])[:, :s * s]
        for i, s in enumerate(_STEPS))
    f = f_flat.reshape(16, 16, _B, _L).transpose(2, 3, 0, 1)

    d = jax.nn.relu(_conv2d(f, p['dec_win'], p['dec_bin']))
    for i in range(4):
        d = jax.image.resize(d, (d.shape[0], d.shape[1],
                                 d.shape[2] * 2, d.shape[3] * 2), 'bilinear')
        d = jax.nn.relu(_conv2d(d, p['dec_w%d' % i], p['dec_b%d' % i],
                                padding=1))
    d = _conv2d(d, p['dec_wout'], p['dec_bout'], padding=1)
    return d, token_maps, vqloss


# speedup vs baseline: 1.0802x; 1.0236x over previous
import jax
import jax.numpy as jnp
from jax.experimental import pallas as pl
from jax.experimental.pallas import tpu as pltpu

_STEPS = (1, 2, 3, 4, 5, 6, 8, 10, 13, 16)
_L = 32
_K = 4096
_B = 8
_M = 256
_BC = _B * _L


def _round8(n):
    return ((n + 7) // 8) * 8


_NPAD = tuple(_round8(s * s) for s in _STEPS)


def _conv2d(x, w, b, stride=1, padding=0):
    out = jax.lax.conv_general_dilated(
        x, w, (stride, stride), [(padding, padding), (padding, padding)],
        dimension_numbers=('NCHW', 'OIHW', 'NCHW'))
    return out + b[None, :, None, None]


def _resize_consts():
    eye16 = jnp.eye(16, dtype=jnp.float32)
    ds, u9s = [], []
    for s, npad in zip(_STEPS, _NPAD):
        n = s * s
        r = jax.image.resize(eye16, (s, 16), 'bilinear')
        d = r[:, None, :, None] * r[None, :, None, :]
        d = d.reshape(n, 256)
        d = jnp.zeros((npad, 256), jnp.float32).at[:n].set(d)
        u = jax.image.resize(jnp.eye(s, dtype=jnp.float32), (16, s), 'bilinear')
        ufull = (u[:, None, :, None] * u[None, :, None, :])
        up = jnp.pad(ufull, ((1, 1), (1, 1), (0, 0), (0, 0)))
        uc = jnp.zeros((256, npad), jnp.float32).at[:, :n].set(
            ufull.reshape(256, n))
        ds.append(d)
        u9s.append(uc)
    return ds, u9s


def _vq_body(xm_ref, ct_ref, w_ref, b_ref, *refs):
    nd = len(_STEPS)
    d_refs = refs[:nd]
    u9_refs = refs[nd:2 * nd]
    f_ref = refs[2 * nd]
    idx_refs = refs[2 * nd + 1:3 * nd + 1]
    loss_ref = refs[3 * nd + 1]

    x_cur = xm_ref[...]
    ct = ct_ref[...]
    ct16 = ct.astype(jnp.bfloat16)
    c2 = jnp.sum(ct * ct, axis=0, keepdims=True)
    f_acc = jnp.zeros((_M, _BC), jnp.float32)
    loss = jnp.zeros((1, 1), jnp.float32)

    for i, s in enumerate(_STEPS):
        n = s * s
        npad = _NPAD[i]
        rows = _B * npad
        dmat = d_refs[i][...]
        zbc = jnp.dot(dmat, x_cur, preferred_element_type=jnp.float32,
                      precision=jax.lax.Precision.HIGHEST)
        zr = zbc.reshape(npad, _B, _L).transpose(1, 0, 2).reshape(rows, _L)
        n_chunks = -(-rows // 512)
        while rows % n_chunks or (rows // n_chunks) % 8:
            n_chunks += 1
        ch = rows // n_chunks
        idx_parts, q_parts = [], []
        for ci in range(n_chunks):
            zc = zr[ci * ch:(ci + 1) * ch]
            g = jnp.dot(zc.astype(jnp.bfloat16), ct16,
                        preferred_element_type=jnp.float32)
            z2 = jnp.sum(zc * zc, axis=1, keepdims=True)
            score = z2 - 2.0 * g + c2
            mn = jnp.min(score, axis=1, keepdims=True)
            kio = jax.lax.broadcasted_iota(jnp.int32, (ch, _K), 1)
            idx_parts.append(jnp.min(jnp.where(score == mn, kio, _K),
                                     axis=1, keepdims=True))
            onehot = (kio == idx_parts[-1]).astype(jnp.float32)
            q_parts.append(jax.lax.dot_general(
                onehot, ct, (((1,), (1,)), ((), ())),
                preferred_element_type=jnp.float32,
                precision=jax.lax.Precision.HIGHEST))
        idx = jnp.concatenate(idx_parts, axis=0) if n_chunks > 1 else idx_parts[0]
        q = jnp.concatenate(q_parts, axis=0) if n_chunks > 1 else q_parts[0]
        err = q - zr
        rmask = (jax.lax.broadcasted_iota(jnp.int32, (rows, _L), 0) % npad) < n
        err = jnp.where(rmask, err, 0.0)
        sq = jnp.sum(jnp.sum(err * err, axis=1, keepdims=True),
                     axis=0, keepdims=True)
        loss = loss + sq * (1.25 / (_B * n * _L))
        q_st = zr + (q - zr)
        qbc = q_st.reshape(_B, npad, _L).transpose(1, 0, 2).reshape(npad, _BC)
        pc = jnp.dot(u9_refs[i][...], qbc,
                     preferred_element_type=jnp.float32,
                     precision=jax.lax.Precision.HIGHEST)
        pp = jnp.concatenate([jnp.zeros((24, _BC), jnp.float32), pc,
                              jnp.zeros((24, _BC), jnp.float32)])
        wio = jax.lax.broadcasted_iota(jnp.int32, (_M, _BC), 0) % 16
        taps = []
        for dy in range(3):
            for dx in range(3):
                sl = pp[24 + (dy - 1) * 16 + (dx - 1):
                        24 + (dy - 1) * 16 + (dx - 1) + _M]
                if dx == 0:
                    sl = jnp.where(wio == 0, 0.0, sl)
                elif dx == 2:
                    sl = jnp.where(wio == 15, 0.0, sl)
                taps.append(sl)
        u9q16 = jnp.concatenate(taps, axis=1).astype(jnp.bfloat16)
        r = jnp.dot(u9q16, w_ref[i],
                    preferred_element_type=jnp.float32)
        r = r + b_ref[i, :][None, :]
        x_cur = x_cur - r
        f_acc = f_acc + r
        idx_refs[i][...] = idx

    f_ref[...] = f_acc
    loss_ref[...] = loss


def _vq_stage(xm, ct, wbd, ballbc, dmats, u9mats):
    whole = lambda a: pl.BlockSpec(a.shape, lambda: (0,) * a.ndim)
    in_specs = [whole(xm), whole(ct), whole(wbd), whole(ballbc)]
    in_specs += [whole(d) for d in dmats]
    in_specs += [whole(u) for u in u9mats]
    out_shape = [jax.ShapeDtypeStruct((_M, _BC), jnp.float32)]
    out_shape += [jax.ShapeDtypeStruct((_B * npad, 1), jnp.int32)
                  for npad in _NPAD]
    out_shape += [jax.ShapeDtypeStruct((1, 1), jnp.float32)]
    out_specs = [pl.BlockSpec((_M, _BC), lambda: (0, 0))]
    out_specs += [pl.BlockSpec((_B * npad, 1), lambda: (0, 0))
                  for npad in _NPAD]
    out_specs += [pl.BlockSpec((1, 1), lambda: (0, 0))]
    return pl.pallas_call(
        _vq_body,
        in_specs=in_specs,
        out_specs=out_specs,
        out_shape=out_shape,
    )(xm, ct, wbd, ballbc, *dmats, *u9mats)


def kernel(x, params):
    p = params
    h = x
    for i in range(4):
        h = jax.nn.relu(_conv2d(h, p['enc_w%d' % i], p['enc_b%d' % i],
                                stride=2, padding=1))
    h = _conv2d(h, p['enc_wout'], p['enc_bout'])
    h = _conv2d(h, p['quant_w'], p['quant_b'])
    x_lat = h.astype(jnp.float32)

    xm = x_lat.transpose(2, 3, 0, 1).reshape(_M, _BC)
    ct = p['codebook'].T
    wtap = jnp.stack([
        jnp.stack([p['res_w%d' % i][:, :, t // 3, t % 3].T for t in range(9)])
        for i in range(len(_STEPS))])
    eyeb = jnp.eye(_B, dtype=jnp.float32)
    wbd = (wtap[:, :, None, :, None, :] * eyeb[None, None, :, None, :, None])
    wbd = wbd.reshape(len(_STEPS), 9 * _BC, _BC).astype(jnp.bfloat16)
    ballbc = jnp.zeros((16, _BC), jnp.float32).at[:len(_STEPS)].set(
        jnp.stack([jnp.tile(p['res_b%d' % i], _B)
                   for i in range(len(_STEPS))]))
    dmats, u9mats = _resize_consts()

    outs = _vq_stage(xm, ct, wbd, ballbc, dmats, u9mats)
    f_flat = outs[0]
    idx_pads = outs[1:1 + len(_STEPS)]
    vqloss = outs[-1][0, 0]

    token_maps = tuple(
        idx_pads[i].reshape(_B, _NPAD[i])[:, :s * s]
        for i, s in enumerate(_STEPS))
    f = f_flat.reshape(16, 16, _B, _L).transpose(2, 3, 0, 1)

    d = jax.nn.relu(_conv2d(f, p['dec_win'], p['dec_bin']))
    for i in range(4):
        d = jax.image.resize(d, (d.shape[0], d.shape[1],
                                 d.shape[2] * 2, d.shape[3] * 2), 'bilinear')
        d = jax.nn.relu(_conv2d(d, p['dec_w%d' % i], p['dec_b%d' % i],
                                padding=1))
    d = _conv2d(d, p['dec_wout'], p['dec_bout'], padding=1)
    return d, token_maps, vqloss
